# baseline (speedup 1.0000x reference)
import jax
import jax.numpy as jnp
from jax.experimental import pallas as pl
from jax.experimental.pallas import tpu as pltpu


def _embed_kernel(ids_ref, embs_ref, out_ref):
    e = embs_ref[...]
    norm = jnp.sqrt(jnp.sum(e * e, axis=1, keepdims=True))
    normed = e / jnp.maximum(norm, 1e-12)
    out_ref[...] = jnp.dot(
        ids_ref[...], normed, preferred_element_type=jnp.float32
    )


def kernel(ids, embs):
    b, v = ids.shape
    _, d = embs.shape
    bm = 2048
    return pl.pallas_call(
        _embed_kernel,
        grid=(b // bm,),
        in_specs=[
            pl.BlockSpec((bm, v), lambda i: (i, 0)),
            pl.BlockSpec((v, d), lambda i: (0, 0)),
        ],
        out_specs=pl.BlockSpec((bm, d), lambda i: (i, 0)),
        out_shape=jax.ShapeDtypeStruct((b, d), jnp.float32),
        compiler_params=pltpu.CompilerParams(
            dimension_semantics=("arbitrary",)
        ),
    )(ids, embs)
